# speedup vs baseline: 5.0886x; 5.0886x over previous
import functools

import jax
import jax.numpy as jnp
from jax import lax
from jax.experimental import pallas as pl
from jax.experimental.pallas import tpu as pltpu
from jax.experimental.pallas import tpu_sc as plsc

D_MODEL = 1024
NUM_CORES = 2
NUM_SUBCORES = 16
NUM_WORKERS = NUM_CORES * NUM_SUBCORES
CHUNK = 64


def _emb_body(tokens_hbm, table_hbm, out_hbm, idx_v, rows_v, sem):
    b_per_w = tokens_hbm.shape[0] // NUM_WORKERS
    wid = lax.axis_index("s") * NUM_CORES + lax.axis_index("c")
    base = wid * b_per_w
    pltpu.sync_copy(tokens_hbm.at[pl.ds(base, b_per_w)], idx_v)
    for c in range(b_per_w // CHUNK):
        pltpu.async_copy(
            table_hbm.at[idx_v.at[pl.ds(c * CHUNK, CHUNK)]], rows_v, sem
        ).wait()
        pltpu.sync_copy(rows_v, out_hbm.at[pl.ds(base + c * CHUNK, CHUNK)])


@jax.jit
def _embed(tokens_flat, weight):
    n = tokens_flat.shape[0]
    run = pl.kernel(
        _emb_body,
        out_type=jax.ShapeDtypeStruct((n, D_MODEL), jnp.float32),
        mesh=plsc.VectorSubcoreMesh(core_axis_name="c", subcore_axis_name="s"),
        scratch_types=[
            pltpu.VMEM((n // NUM_WORKERS,), jnp.int32),
            pltpu.VMEM((CHUNK, D_MODEL), jnp.float32),
            pltpu.SemaphoreType.DMA,
        ],
    )
    return run(tokens_flat, weight)


def kernel(tokens, weight):
    b, s = tokens.shape
    flat = tokens.reshape(b * s).astype(jnp.int32)
    out = _embed(flat, weight)
    return out.reshape(b, s, D_MODEL)


# speedup vs baseline: 5.2242x; 1.0267x over previous
import functools

import jax
import jax.numpy as jnp
from jax import lax
from jax.experimental import pallas as pl
from jax.experimental.pallas import tpu as pltpu
from jax.experimental.pallas import tpu_sc as plsc

D_MODEL = 1024
NUM_CORES = 2
NUM_SUBCORES = 16
NUM_WORKERS = NUM_CORES * NUM_SUBCORES
CHUNK = 32


def _emb_body(tokens_hbm, table_hbm, out_hbm, idx_v, rows0, rows1, gsem, osem):
    b_per_w = tokens_hbm.shape[0] // NUM_WORKERS
    nchunks = b_per_w // CHUNK
    wid = lax.axis_index("s") * NUM_CORES + lax.axis_index("c")
    base = wid * b_per_w
    bufs = (rows0, rows1)
    pltpu.sync_copy(tokens_hbm.at[pl.ds(base, b_per_w)], idx_v)

    def gather(c, buf):
        return pltpu.async_copy(
            table_hbm.at[idx_v.at[pl.ds(c * CHUNK, CHUNK)]], buf, gsem
        )

    def put(c, buf):
        return pltpu.async_copy(buf, out_hbm.at[pl.ds(base + c * CHUNK, CHUNK)], osem)

    g = gather(0, bufs[0])
    pending = []
    for c in range(nchunks):
        g.wait()
        pending.append(put(c, bufs[c % 2]))
        if c + 1 < nchunks:
            if len(pending) >= 2:
                pending.pop(0).wait()
            g = gather(c + 1, bufs[(c + 1) % 2])
    for o in pending:
        o.wait()


@jax.jit
def _embed(tokens_flat, weight):
    n = tokens_flat.shape[0]
    run = pl.kernel(
        _emb_body,
        out_type=jax.ShapeDtypeStruct((n, D_MODEL), jnp.float32),
        mesh=plsc.VectorSubcoreMesh(core_axis_name="c", subcore_axis_name="s"),
        scratch_types=[
            pltpu.VMEM((n // NUM_WORKERS,), jnp.int32),
            pltpu.VMEM((CHUNK, D_MODEL), jnp.float32),
            pltpu.VMEM((CHUNK, D_MODEL), jnp.float32),
            pltpu.SemaphoreType.DMA,
            pltpu.SemaphoreType.DMA,
        ],
    )
    return run(tokens_flat, weight)


def kernel(tokens, weight):
    b, s = tokens.shape
    flat = tokens.reshape(b * s).astype(jnp.int32)
    out = _embed(flat, weight)
    return out.reshape(b, s, D_MODEL)


# speedup vs baseline: 5.4589x; 1.0449x over previous
import functools

import jax
import jax.numpy as jnp
from jax import lax
from jax.experimental import pallas as pl
from jax.experimental.pallas import tpu as pltpu
from jax.experimental.pallas import tpu_sc as plsc

D_MODEL = 1024
NUM_CORES = 2
NUM_SUBCORES = 16
NUM_WORKERS = NUM_CORES * NUM_SUBCORES
CHUNK = 32


NBUF = 3


def _emb_body(tokens_hbm, table_hbm, out_hbm, idx_v, rows0, rows1, rows2, gsem, osem):
    b_per_w = tokens_hbm.shape[0] // NUM_WORKERS
    nchunks = b_per_w // CHUNK
    wid = lax.axis_index("s") * NUM_CORES + lax.axis_index("c")
    base = wid * b_per_w
    bufs = (rows0, rows1, rows2)
    pltpu.sync_copy(tokens_hbm.at[pl.ds(base, b_per_w)], idx_v)

    def gather(c, buf):
        return pltpu.async_copy(
            table_hbm.at[idx_v.at[pl.ds(c * CHUNK, CHUNK)]], buf, gsem
        )

    def put(c, buf):
        return pltpu.async_copy(buf, out_hbm.at[pl.ds(base + c * CHUNK, CHUNK)], osem)

    grefs = {}
    for c in range(min(NBUF - 1, nchunks)):
        grefs[c] = gather(c, bufs[c % NBUF])
    pending = []
    for c in range(nchunks):
        grefs.pop(c).wait()
        pending.append(put(c, bufs[c % NBUF]))
        nxt = c + NBUF - 1
        if nxt < nchunks:
            if nxt >= NBUF:
                pending.pop(0).wait()
            grefs[nxt] = gather(nxt, bufs[nxt % NBUF])
    for o in pending:
        o.wait()


@jax.jit
def _embed(tokens_flat, weight):
    n = tokens_flat.shape[0]
    run = pl.kernel(
        _emb_body,
        out_type=jax.ShapeDtypeStruct((n, D_MODEL), jnp.float32),
        mesh=plsc.VectorSubcoreMesh(core_axis_name="c", subcore_axis_name="s"),
        scratch_types=[
            pltpu.VMEM((n // NUM_WORKERS,), jnp.int32),
            pltpu.VMEM((CHUNK, D_MODEL), jnp.float32),
            pltpu.VMEM((CHUNK, D_MODEL), jnp.float32),
            pltpu.VMEM((CHUNK, D_MODEL), jnp.float32),
            pltpu.SemaphoreType.DMA,
            pltpu.SemaphoreType.DMA,
        ],
    )
    return run(tokens_flat, weight)


def kernel(tokens, weight):
    b, s = tokens.shape
    flat = tokens.reshape(b * s).astype(jnp.int32)
    out = _embed(flat, weight)
    return out.reshape(b, s, D_MODEL)
